# Initial kernel scaffold; baseline (speedup 1.0000x reference)
import jax
import jax.numpy as jnp
from jax.experimental import pallas as pl


def kernel(x, concepts):
    raise NotImplementedError("write your pallas kernel here")


# speedup vs baseline: 3.1294x; 3.1294x over previous
import jax
import jax.numpy as jnp
from jax.experimental import pallas as pl

_NUM_INTERVENTIONS = 128


def _select_body(mask_ref, x_ref, c_ref, o_ref):
    o_ref[...] = jnp.where(mask_ref[...] != 0, c_ref[...], x_ref[...])


def kernel(x, concepts):
    n, d = x.shape
    perm = jax.random.permutation(jax.random.key(42), d)
    idx = perm[:_NUM_INTERVENTIONS]
    mask = jnp.zeros((1, d), jnp.int32).at[0, idx].set(1)

    block = 2048
    grid = (n // block,)
    return pl.pallas_call(
        _select_body,
        grid=grid,
        in_specs=[
            pl.BlockSpec((1, d), lambda i: (0, 0)),
            pl.BlockSpec((block, d), lambda i: (i, 0)),
            pl.BlockSpec((block, d), lambda i: (i, 0)),
        ],
        out_specs=pl.BlockSpec((block, d), lambda i: (i, 0)),
        out_shape=jax.ShapeDtypeStruct((n, d), x.dtype),
    )(mask, x, concepts)
